# Initial kernel scaffold; baseline (speedup 1.0000x reference)
import jax
import jax.numpy as jnp
from jax.experimental import pallas as pl


def kernel(nodes, edge_dst, emb_uv, rep, emb_r, W1, b1, W2, b2, A1, a1, A2, a2, A3, a3):
    raise NotImplementedError("write your pallas kernel here")


# speedup vs baseline: 2.5545x; 2.5545x over previous
import functools
import jax
import jax.numpy as jnp
from jax.experimental import pallas as pl
from jax.experimental.pallas import tpu as pltpu

NUM_NODES = 10000
D = 128
BLK = 1280


def _stage1_body(uv_ref, er_ref, rep_ref,
                 W1u_ref, W1r_ref, b1_ref, W2_ref, b2_ref,
                 A1o_ref, A1rep_ref, a1_ref, A2_ref, a2_ref, A3_ref, a3_ref,
                 oh_ref, s_ref, m_ref):
    uv = uv_ref[...]
    er = er_ref[...]
    rep = rep_ref[...]
    f32 = jnp.float32
    h = jnp.maximum(
        jnp.dot(uv, W1u_ref[...], preferred_element_type=f32)
        + jnp.dot(er, W1r_ref[...], preferred_element_type=f32)
        + b1_ref[...], 0.0)
    oh = jnp.maximum(jnp.dot(h, W2_ref[...], preferred_element_type=f32)
                     + b2_ref[...], 0.0)
    x1 = jnp.maximum(
        jnp.dot(oh, A1o_ref[...], preferred_element_type=f32)
        + jnp.dot(rep, A1rep_ref[...], preferred_element_type=f32)
        + a1_ref[...], 0.0)
    x2 = jnp.maximum(jnp.dot(x1, A2_ref[...], preferred_element_type=f32)
                     + a2_ref[...], 0.0)
    s = jnp.sum(x2 * A3_ref[...], axis=1, keepdims=True) + a3_ref[...]
    oh_ref[...] = oh
    s_ref[...] = s

    i = pl.program_id(0)

    @pl.when(i == 0)
    def _():
        m_ref[...] = jnp.full_like(m_ref, -jnp.inf)

    m_ref[...] = jnp.maximum(m_ref[...], jnp.full((1, 1), jnp.max(s)))


def _stage1(emb_uv, emb_r, rep, W1, b1, W2, b2, A1, a1, A2, a2, A3, a3):
    E = emb_uv.shape[0]
    grid = E // BLK
    blk_e = lambda i: (i, 0)
    blk_w = lambda i: (0, 0)
    edge_spec = pl.BlockSpec((BLK, D), blk_e)
    w_spec = pl.BlockSpec((D, D), blk_w)
    b_spec = pl.BlockSpec((1, D), blk_w)
    return pl.pallas_call(
        _stage1_body,
        grid=(grid,),
        in_specs=[edge_spec, edge_spec, edge_spec,
                  w_spec, w_spec, b_spec, w_spec, b_spec,
                  w_spec, w_spec, b_spec, w_spec, b_spec,
                  b_spec, pl.BlockSpec((1, 1), blk_w)],
        out_specs=[edge_spec,
                   pl.BlockSpec((BLK, 1), blk_e),
                   pl.BlockSpec((1, 1), blk_w)],
        out_shape=[jax.ShapeDtypeStruct((E, D), jnp.float32),
                   jax.ShapeDtypeStruct((E, 1), jnp.float32),
                   jax.ShapeDtypeStruct((1, 1), jnp.float32)],
    )(emb_uv, emb_r, rep,
      W1[:D], W1[D:], b1.reshape(1, D), W2, b2.reshape(1, D),
      A1[:D], A1[D:], a1.reshape(1, D), A2, a2.reshape(1, D),
      A3.reshape(1, D), a3.reshape(1, 1))


def kernel(nodes, edge_dst, emb_uv, rep, emb_r, W1, b1, W2, b2, A1, a1, A2, a2, A3, a3):
    oh, s, m = _stage1(emb_uv, emb_r, rep, W1, b1, W2, b2, A1, a1, A2, a2, A3, a3)
    s1 = s[:, 0]
    e = jnp.exp(s1 - m[0, 0])
    z = jax.ops.segment_sum(e, edge_dst, num_segments=NUM_NODES)
    feat = jax.ops.segment_sum(oh * e[:, None], edge_dst, num_segments=NUM_NODES)
    feat = feat / (z[:, None] + 1e-16)
    return feat[nodes]


# speedup vs baseline: 4.0937x; 1.6025x over previous
import functools
import jax
import jax.numpy as jnp
from jax import lax
from jax.experimental import pallas as pl
from jax.experimental.pallas import tpu as pltpu
from jax.experimental.pallas import tpu_sc as plsc

NUM_NODES = 10000
D = 128
DZ = D + 16
BLK = 1280
NTILES = 32
CH = 80
NODES_PAD = 10240
ROWS_PER_TILE = NODES_PAD // 16


def _stage1_body(uv_ref, er_ref, rep_ref,
                 W1u_ref, W1r_ref, b1_ref, W2_ref, b2_ref,
                 A1o_ref, A1rep_ref, a1_ref, A2_ref, a2_ref, A3_ref, a3_ref,
                 oh_ref, s_ref, m_ref):
    uv = uv_ref[...]
    er = er_ref[...]
    rep = rep_ref[...]
    f32 = jnp.float32
    h = jnp.maximum(
        jnp.dot(uv, W1u_ref[...], preferred_element_type=f32)
        + jnp.dot(er, W1r_ref[...], preferred_element_type=f32)
        + b1_ref[...], 0.0)
    oh = jnp.maximum(jnp.dot(h, W2_ref[...], preferred_element_type=f32)
                     + b2_ref[...], 0.0)
    x1 = jnp.maximum(
        jnp.dot(oh, A1o_ref[...], preferred_element_type=f32)
        + jnp.dot(rep, A1rep_ref[...], preferred_element_type=f32)
        + a1_ref[...], 0.0)
    x2 = jnp.maximum(jnp.dot(x1, A2_ref[...], preferred_element_type=f32)
                     + a2_ref[...], 0.0)
    s = jnp.sum(x2 * A3_ref[...], axis=1, keepdims=True) + a3_ref[...]
    oh_ref[...] = oh
    s_ref[...] = s

    i = pl.program_id(0)

    @pl.when(i == 0)
    def _():
        m_ref[...] = jnp.full_like(m_ref, -jnp.inf)

    m_ref[...] = jnp.maximum(m_ref[...], jnp.full((1, 1), jnp.max(s)))


def _stage1(emb_uv, emb_r, rep, W1, b1, W2, b2, A1, a1, A2, a2, A3, a3):
    E = emb_uv.shape[0]
    grid = E // BLK
    blk_e = lambda i: (i, 0)
    blk_w = lambda i: (0, 0)
    edge_spec = pl.BlockSpec((BLK, D), blk_e)
    w_spec = pl.BlockSpec((D, D), blk_w)
    b_spec = pl.BlockSpec((1, D), blk_w)
    return pl.pallas_call(
        _stage1_body,
        grid=(grid,),
        in_specs=[edge_spec, edge_spec, edge_spec,
                  w_spec, w_spec, b_spec, w_spec, b_spec,
                  w_spec, w_spec, b_spec, w_spec, b_spec,
                  b_spec, pl.BlockSpec((1, 1), blk_w)],
        out_specs=[edge_spec,
                   pl.BlockSpec((BLK, 1), blk_e),
                   pl.BlockSpec((1, 1), blk_w)],
        out_shape=[jax.ShapeDtypeStruct((E, D), jnp.float32),
                   jax.ShapeDtypeStruct((E, 1), jnp.float32),
                   jax.ShapeDtypeStruct((1, 1), jnp.float32)],
    )(emb_uv, emb_r, rep,
      W1[:D], W1[D:], b1.reshape(1, D), W2, b2.reshape(1, D),
      A1[:D], A1[D:], a1.reshape(1, D), A2, a2.reshape(1, D),
      A3.reshape(1, D), a3.reshape(1, 1))


def _stage2_body(s_hbm, dst_hbm, oh_hbm, m_hbm, zeros_hbm, out_hbm,
                 feat_sh, idx_v, s_v, e_v, oh_v, msg_v, m_v):
    cid = lax.axis_index("c")
    sid = lax.axis_index("s")
    wid = cid * 16 + sid
    nchunk = (s_hbm.shape[0] // NTILES) // CH

    pltpu.sync_copy(zeros_hbm, feat_sh.at[pl.ds(sid * ROWS_PER_TILE, ROWS_PER_TILE)])
    pltpu.sync_copy(m_hbm, m_v)
    plsc.subcore_barrier()

    m_vec = m_v[...]
    col128 = jnp.full((16,), D, dtype=jnp.int32)
    lane = jnp.arange(16, dtype=jnp.int32)

    def _zero_tail(i, _):
        msg_v[i, pl.ds(D, 16)] = jnp.zeros((16,), jnp.float32)
        return 0
    lax.fori_loop(0, CH, _zero_tail, 0)

    def _chunk(k, _):
        row = wid * nchunk + k
        e0 = row * CH
        pltpu.sync_copy(dst_hbm.at[row], idx_v.at[0])
        pltpu.sync_copy(s_hbm.at[pl.ds(e0, CH)], s_v)
        pltpu.sync_copy(oh_hbm.at[pl.ds(e0, CH)], oh_v)
        for g in range(CH // 16):
            e16 = jnp.exp(s_v[pl.ds(16 * g, 16)] - m_vec)
            e_v[pl.ds(16 * g, 16)] = e16
            plsc.store_scatter(msg_v, [lane + 16 * g, col128], e16)
        def _scale(i, _):
            esp = plsc.load_gather(e_v, [jnp.full((16,), i, dtype=jnp.int32)])
            for j in range(D // 16):
                msg_v[i, pl.ds(16 * j, 16)] = oh_v[i, pl.ds(16 * j, 16)] * esp
            return 0
        lax.fori_loop(0, CH, _scale, 0)
        pltpu.sync_copy(msg_v, feat_sh.at[idx_v.at[0]], add=True)
        return 0

    lax.fori_loop(0, nchunk, _chunk, 0)

    plsc.subcore_barrier()
    r0 = sid * ROWS_PER_TILE
    pltpu.sync_copy(feat_sh.at[pl.ds(r0, ROWS_PER_TILE)],
                    out_hbm.at[cid, pl.ds(r0, ROWS_PER_TILE)])


def _stage3_body(pa_hbm, pb_hbm, nodes_hbm, out_hbm, nidx_v, ra_v, rb_v, o_v, sem):
    cid = lax.axis_index("c")
    sid = lax.axis_index("s")
    wid = cid * 16 + sid
    NB = 128
    n_per_tile = nodes_hbm.shape[0] // NTILES
    col128 = jnp.full((16,), D, dtype=jnp.int32)

    for c in range(n_per_tile // NB):
        base = wid * n_per_tile + c * NB
        pltpu.sync_copy(nodes_hbm.at[pl.ds(base, NB)], nidx_v.at[0])
        pltpu.async_copy(pa_hbm.at[nidx_v.at[0]], ra_v, sem).wait()
        pltpu.async_copy(pb_hbm.at[nidx_v.at[0]], rb_v, sem).wait()

        def _row(i, _):
            isp = jnp.full((16,), i, dtype=jnp.int32)
            za = plsc.load_gather(ra_v, [isp, col128])
            zb = plsc.load_gather(rb_v, [isp, col128])
            inv = 1.0 / (za + zb + 1e-16)
            for j in range(D // 16):
                sl = pl.ds(16 * j, 16)
                o_v[i, sl] = (ra_v[i, sl] + rb_v[i, sl]) * inv
            return 0
        lax.fori_loop(0, NB, _row, 0)
        pltpu.sync_copy(o_v, out_hbm.at[pl.ds(base, NB)])


def _sc_mesh():
    return plsc.VectorSubcoreMesh(core_axis_name="c", subcore_axis_name="s")


def _stage2(s_flat, dst2d, oh, m16, zeros):
    return pl.kernel(
        _stage2_body,
        out_type=jax.ShapeDtypeStruct((2, NODES_PAD, DZ), jnp.float32),
        mesh=_sc_mesh(),
        compiler_params=pltpu.CompilerParams(use_tc_tiling_on_sc=False, needs_layout_passes=False),
        scratch_types=[
            pltpu.VMEM_SHARED((NODES_PAD, DZ), jnp.float32),
            pltpu.VMEM((1, CH), jnp.int32),
            pltpu.VMEM((CH,), jnp.float32),
            pltpu.VMEM((CH,), jnp.float32),
            pltpu.VMEM((CH, D), jnp.float32),
            pltpu.VMEM((CH, DZ), jnp.float32),
            pltpu.VMEM((16,), jnp.float32),
        ],
    )(s_flat, dst2d, oh, m16, zeros)


def _stage3(pa, pb, nodes):
    return pl.kernel(
        _stage3_body,
        out_type=jax.ShapeDtypeStruct((8192, D), jnp.float32),
        mesh=_sc_mesh(),
        compiler_params=pltpu.CompilerParams(use_tc_tiling_on_sc=False, needs_layout_passes=False),
        scratch_types=[
            pltpu.VMEM((1, 128), jnp.int32),
            pltpu.VMEM((128, DZ), jnp.float32),
            pltpu.VMEM((128, DZ), jnp.float32),
            pltpu.VMEM((128, D), jnp.float32),
            pltpu.SemaphoreType.DMA,
        ],
    )(pa, pb, nodes)


def kernel(nodes, edge_dst, emb_uv, rep, emb_r, W1, b1, W2, b2, A1, a1, A2, a2, A3, a3):
    E = emb_uv.shape[0]
    oh, s, m = _stage1(emb_uv, emb_r, rep, W1, b1, W2, b2, A1, a1, A2, a2, A3, a3)
    s_flat = s.reshape(E)
    dst2d = edge_dst.astype(jnp.int32).reshape(E // CH, CH)
    m16 = jnp.broadcast_to(m.reshape(()), (16,))
    zeros = jnp.zeros((ROWS_PER_TILE, DZ), jnp.float32)
    partials = _stage2(s_flat, dst2d, oh, m16, zeros)
    return _stage3(partials[0], partials[1], nodes.astype(jnp.int32))


# speedup vs baseline: 6.5758x; 1.6063x over previous
import jax
import jax.numpy as jnp
from jax import lax
from jax.experimental import pallas as pl
from jax.experimental.pallas import tpu as pltpu
from jax.experimental.pallas import tpu_sc as plsc

NUM_NODES = 10000
D = 128
ZW = 16
BLK = 1280
NTILES = 32
CH = 80
NODES_PAD = 10112
ROWS_PER_TILE = NODES_PAD // 16


def _stage1_body(uv_ref, er_ref, rep_ref,
                 W1u_ref, W1r_ref, b1_ref, W2_ref, b2_ref,
                 A1o_ref, A1rep_ref, a1_ref, A2_ref, a2_ref, A3_ref, a3_ref,
                 oh_ref, s_ref, m_ref):
    uv = uv_ref[...]
    er = er_ref[...]
    rep = rep_ref[...]
    f32 = jnp.float32
    h = jnp.maximum(
        jnp.dot(uv, W1u_ref[...], preferred_element_type=f32)
        + jnp.dot(er, W1r_ref[...], preferred_element_type=f32)
        + b1_ref[...], 0.0)
    oh = jnp.maximum(jnp.dot(h, W2_ref[...], preferred_element_type=f32)
                     + b2_ref[...], 0.0)
    x1 = jnp.maximum(
        jnp.dot(oh, A1o_ref[...], preferred_element_type=f32)
        + jnp.dot(rep, A1rep_ref[...], preferred_element_type=f32)
        + a1_ref[...], 0.0)
    x2 = jnp.maximum(jnp.dot(x1, A2_ref[...], preferred_element_type=f32)
                     + a2_ref[...], 0.0)
    s = jnp.sum(x2 * A3_ref[...], axis=1, keepdims=True) + a3_ref[...]
    oh_ref[...] = oh
    s_ref[...] = s

    i = pl.program_id(0)

    @pl.when(i == 0)
    def _():
        m_ref[...] = jnp.full_like(m_ref, -jnp.inf)

    m_ref[...] = jnp.maximum(m_ref[...], jnp.full((1, 1), jnp.max(s)))


def _stage1(emb_uv, emb_r, rep, W1, b1, W2, b2, A1, a1, A2, a2, A3, a3):
    E = emb_uv.shape[0]
    grid = E // BLK
    blk_e = lambda i: (i, 0)
    blk_w = lambda i: (0, 0)
    edge_spec = pl.BlockSpec((BLK, D), blk_e)
    w_spec = pl.BlockSpec((D, D), blk_w)
    b_spec = pl.BlockSpec((1, D), blk_w)
    return pl.pallas_call(
        _stage1_body,
        grid=(grid,),
        in_specs=[edge_spec, edge_spec, edge_spec,
                  w_spec, w_spec, b_spec, w_spec, b_spec,
                  w_spec, w_spec, b_spec, w_spec, b_spec,
                  b_spec, pl.BlockSpec((1, 1), blk_w)],
        out_specs=[edge_spec,
                   pl.BlockSpec((BLK, 1), blk_e),
                   pl.BlockSpec((1, 1), blk_w)],
        out_shape=[jax.ShapeDtypeStruct((E, D), jnp.float32),
                   jax.ShapeDtypeStruct((E, 1), jnp.float32),
                   jax.ShapeDtypeStruct((1, 1), jnp.float32)],
    )(emb_uv, emb_r, rep,
      W1[:D], W1[D:], b1.reshape(1, D), W2, b2.reshape(1, D),
      A1[:D], A1[D:], a1.reshape(1, D), A2, a2.reshape(1, D),
      A3.reshape(1, D), a3.reshape(1, 1))


def _stage2_body(s_hbm, dst_hbm, oh_hbm, m_hbm, zf_hbm, zz_hbm,
                 outf_hbm, outz_hbm,
                 feat_sh, z_sh,
                 idx0, s0, oh0, mz0, idx1, s1, oh1, mz1,
                 e_v, m_v, gsem0, gsem1, ssem0, ssem1):
    cid = lax.axis_index("c")
    sid = lax.axis_index("s")
    wid = cid * 16 + sid
    nchunk = (s_hbm.shape[0] // NTILES) // CH

    r0 = sid * ROWS_PER_TILE
    pltpu.sync_copy(zf_hbm, feat_sh.at[pl.ds(r0, ROWS_PER_TILE)])
    pltpu.sync_copy(zz_hbm, z_sh.at[pl.ds(r0, ROWS_PER_TILE)])
    pltpu.sync_copy(m_hbm, m_v)
    plsc.subcore_barrier()

    m_vec = m_v[...]
    lane = jnp.arange(16, dtype=jnp.int32)
    mask01 = jnp.where(lane == 0, 1.0, 0.0).astype(jnp.float32)
    bufs = [(idx0, s0, oh0, mz0, gsem0, ssem0),
            (idx1, s1, oh1, mz1, gsem1, ssem1)]

    def start_gather(c, b):
        idx, s_b, oh_b, _, gs, _ = bufs[b]
        pltpu.async_copy(dst_hbm.at[c], idx.at[0], gs)
        pltpu.async_copy(s_hbm.at[pl.ds(c * CH, CH)], s_b, gs)
        pltpu.async_copy(oh_hbm.at[pl.ds(c * CH, CH)], oh_b, gs)

    def wait_gather(c, b):
        idx, s_b, oh_b, _, gs, _ = bufs[b]
        pltpu.make_async_copy(dst_hbm.at[c], idx.at[0], gs).wait()
        pltpu.make_async_copy(s_hbm.at[pl.ds(c * CH, CH)], s_b, gs).wait()
        pltpu.make_async_copy(oh_hbm.at[pl.ds(c * CH, CH)], oh_b, gs).wait()

    def start_scatter(b):
        idx, _, oh_b, mz, _, ss = bufs[b]
        pltpu.async_copy(oh_b, feat_sh.at[idx.at[0]], ss, add=True)
        pltpu.async_copy(mz, z_sh.at[idx.at[0]], ss, add=True)

    def wait_scatter(b):
        idx, _, oh_b, mz, _, ss = bufs[b]
        pltpu.make_async_copy(oh_b, feat_sh.at[idx.at[0]], ss).wait()
        pltpu.make_async_copy(mz, z_sh.at[idx.at[0]], ss).wait()

    def compute(b):
        _, s_b, oh_b, mz, _, _ = bufs[b]
        for g in range(CH // 16):
            e_v[pl.ds(16 * g, 16)] = jnp.exp(s_b[pl.ds(16 * g, 16)] - m_vec)

        def _scale8(g8, _):
            for r in range(8):
                i = g8 * 8 + r
                esp = plsc.load_gather(e_v, [jnp.full((16,), i, dtype=jnp.int32)])
                for j in range(D // 16):
                    sl = pl.ds(16 * j, 16)
                    oh_b[i, sl] = oh_b[i, sl] * esp
                mz[i, pl.ds(0, 16)] = esp * mask01
            return 0
        lax.fori_loop(0, CH // 8, _scale8, 0)

    c_base = wid * nchunk
    start_gather(c_base, 0)

    def _pipe(t, _):
        cA = c_base + 2 * t
        wait_gather(cA, 0)
        start_gather(cA + 1, 1)
        compute(0)
        start_scatter(0)
        wait_gather(cA + 1, 1)
        wait_scatter(0)
        start_gather(cA + 2, 0)
        compute(1)
        start_scatter(1)
        wait_scatter(1)
        return 0
    lax.fori_loop(0, (nchunk - 1) // 2, _pipe, 0)

    cE = c_base + nchunk - 1
    wait_gather(cE, 0)
    compute(0)
    start_scatter(0)
    wait_scatter(0)

    plsc.subcore_barrier()
    pltpu.sync_copy(feat_sh.at[pl.ds(r0, ROWS_PER_TILE)],
                    outf_hbm.at[cid, pl.ds(r0, ROWS_PER_TILE)])
    pltpu.sync_copy(z_sh.at[pl.ds(r0, ROWS_PER_TILE)],
                    outz_hbm.at[cid, pl.ds(r0, ROWS_PER_TILE)])


def _stage3_body(pa_hbm, za_hbm, pb_hbm, zb_hbm, nodes_hbm, out_hbm,
                 nidx_v, ra_v, rza_v, rb_v, rzb_v, o_v, sem):
    cid = lax.axis_index("c")
    sid = lax.axis_index("s")
    wid = cid * 16 + sid
    NB = 128
    n_per_tile = nodes_hbm.shape[0] // NTILES
    col0 = jnp.zeros((16,), dtype=jnp.int32)

    for c in range(n_per_tile // NB):
        base = wid * n_per_tile + c * NB
        pltpu.sync_copy(nodes_hbm.at[pl.ds(base, NB)], nidx_v.at[0])
        pltpu.async_copy(pa_hbm.at[nidx_v.at[0]], ra_v, sem)
        pltpu.async_copy(za_hbm.at[nidx_v.at[0]], rza_v, sem)
        pltpu.async_copy(pb_hbm.at[nidx_v.at[0]], rb_v, sem)
        pltpu.async_copy(zb_hbm.at[nidx_v.at[0]], rzb_v, sem)
        pltpu.make_async_copy(pa_hbm.at[nidx_v.at[0]], ra_v, sem).wait()
        pltpu.make_async_copy(za_hbm.at[nidx_v.at[0]], rza_v, sem).wait()
        pltpu.make_async_copy(pb_hbm.at[nidx_v.at[0]], rb_v, sem).wait()
        pltpu.make_async_copy(zb_hbm.at[nidx_v.at[0]], rzb_v, sem).wait()

        def _row(i, _):
            isp = jnp.full((16,), i, dtype=jnp.int32)
            za = plsc.load_gather(rza_v, [isp, col0])
            zb = plsc.load_gather(rzb_v, [isp, col0])
            inv = 1.0 / (za + zb + 1e-16)
            for j in range(D // 16):
                sl = pl.ds(16 * j, 16)
                o_v[i, sl] = (ra_v[i, sl] + rb_v[i, sl]) * inv
            return 0
        lax.fori_loop(0, NB, _row, 0)
        pltpu.sync_copy(o_v, out_hbm.at[pl.ds(base, NB)])


def _sc_mesh():
    return plsc.VectorSubcoreMesh(core_axis_name="c", subcore_axis_name="s")


def _sc_params():
    return pltpu.CompilerParams(use_tc_tiling_on_sc=False,
                                needs_layout_passes=False)


def _stage2(s_flat, dst2d, oh, m16, zf, zz):
    return pl.kernel(
        _stage2_body,
        out_type=[jax.ShapeDtypeStruct((2, NODES_PAD, D), jnp.float32),
                  jax.ShapeDtypeStruct((2, NODES_PAD, ZW), jnp.float32)],
        mesh=_sc_mesh(),
        compiler_params=_sc_params(),
        scratch_types=[
            pltpu.VMEM_SHARED((NODES_PAD, D), jnp.float32),
            pltpu.VMEM_SHARED((NODES_PAD, ZW), jnp.float32),
            pltpu.VMEM((1, CH), jnp.int32),
            pltpu.VMEM((CH,), jnp.float32),
            pltpu.VMEM((CH, D), jnp.float32),
            pltpu.VMEM((CH, ZW), jnp.float32),
            pltpu.VMEM((1, CH), jnp.int32),
            pltpu.VMEM((CH,), jnp.float32),
            pltpu.VMEM((CH, D), jnp.float32),
            pltpu.VMEM((CH, ZW), jnp.float32),
            pltpu.VMEM((CH,), jnp.float32),
            pltpu.VMEM((16,), jnp.float32),
            pltpu.SemaphoreType.DMA,
            pltpu.SemaphoreType.DMA,
            pltpu.SemaphoreType.DMA,
            pltpu.SemaphoreType.DMA,
        ],
    )(s_flat, dst2d, oh, m16, zf, zz)


def _stage3(pa, za, pb, zb, nodes):
    return pl.kernel(
        _stage3_body,
        out_type=jax.ShapeDtypeStruct((8192, D), jnp.float32),
        mesh=_sc_mesh(),
        compiler_params=_sc_params(),
        scratch_types=[
            pltpu.VMEM((1, 128), jnp.int32),
            pltpu.VMEM((128, D), jnp.float32),
            pltpu.VMEM((128, ZW), jnp.float32),
            pltpu.VMEM((128, D), jnp.float32),
            pltpu.VMEM((128, ZW), jnp.float32),
            pltpu.VMEM((128, D), jnp.float32),
            pltpu.SemaphoreType.DMA,
        ],
    )(pa, za, pb, zb, nodes)


def kernel(nodes, edge_dst, emb_uv, rep, emb_r, W1, b1, W2, b2, A1, a1, A2, a2, A3, a3):
    E = emb_uv.shape[0]
    oh, s, m = _stage1(emb_uv, emb_r, rep, W1, b1, W2, b2, A1, a1, A2, a2, A3, a3)
    s_flat = s.reshape(E)
    dst2d = edge_dst.astype(jnp.int32).reshape(E // CH, CH)
    m16 = jnp.broadcast_to(m.reshape(()), (16,))
    zf = jnp.zeros((ROWS_PER_TILE, D), jnp.float32)
    zz = jnp.zeros((ROWS_PER_TILE, ZW), jnp.float32)
    pf, pz = _stage2(s_flat, dst2d, oh, m16, zf, zz)
    return _stage3(pf[0], pz[0], pf[1], pz[1], nodes.astype(jnp.int32))
